# baseline (device time: 99556 ns/iter reference)
import jax
import jax.numpy as jnp
from jax import lax
from jax.experimental import pallas as pl
from jax.experimental.pallas import tpu as pltpu

N_DEV = 4


def kernel(x, router_W, route_idx, expert_W):
    n, d = x.shape
    n_loc, _, h = expert_W.shape
    n_exp = N_DEV * n_loc
    chunk = n // N_DEV
    n_hops = N_DEV - 1

    def body(x_ref, rw_ref, idx_ref, ew_ref, out_ref, comm_ref, send_sems, recv_sems):
        my = lax.axis_index("i")
        left = (my - 1) % N_DEV
        right = (my + 1) % N_DEV

        barrier_sem = pltpu.get_barrier_semaphore()
        for nbr in (left, right):
            pl.semaphore_signal(
                barrier_sem, inc=1,
                device_id=(nbr,), device_id_type=pl.DeviceIdType.MESH,
            )
        pl.semaphore_wait(barrier_sem, 2)

        scores = jnp.dot(x_ref[:, :], rw_ref[:, :],
                         preferred_element_type=jnp.float32)
        m = jnp.max(scores, axis=-1, keepdims=True)
        p = jnp.exp(scores - m)
        p = p / jnp.sum(p, axis=-1, keepdims=True)
        idx0 = idx_ref[:, 0:1]
        idx1 = idx_ref[:, 1:2]
        e_iota = lax.broadcasted_iota(jnp.int32, (n, n_exp), 1)
        g0 = jnp.sum(jnp.where(idx0 == e_iota, p, 0.0), axis=-1, keepdims=True)
        g1 = jnp.sum(jnp.where(idx1 == e_iota, p, 0.0), axis=-1, keepdims=True)
        gs = g0 + g1

        acc = jnp.zeros((n, h), jnp.float32)
        for le in range(n_loc):
            gid = my * n_loc + le
            w = (jnp.where(idx0 == gid, g0 / gs, 0.0)
                 + jnp.where(idx1 == gid, g1 / gs, 0.0))
            acc = acc + jnp.dot(x_ref[:, :] * w, ew_ref[le, :, :],
                                preferred_element_type=jnp.float32)
        out_ref[:, :] = acc

        for hop in range(n_hops):
            send_c = (my - hop) % N_DEV
            recv_c = (my - hop - 1) % N_DEV
            rdma = pltpu.make_async_remote_copy(
                src_ref=out_ref.at[pl.ds(send_c * chunk, chunk), :],
                dst_ref=comm_ref.at[hop],
                send_sem=send_sems.at[hop],
                recv_sem=recv_sems.at[hop],
                device_id=(right,),
                device_id_type=pl.DeviceIdType.MESH,
            )
            rdma.start()
            rdma.wait()
            out_ref[pl.ds(recv_c * chunk, chunk), :] = (
                out_ref[pl.ds(recv_c * chunk, chunk), :] + comm_ref[hop]
            )

        for hop in range(n_hops):
            send_c = (my + 1 - hop) % N_DEV
            rdma = pltpu.make_async_remote_copy(
                src_ref=out_ref.at[pl.ds(send_c * chunk, chunk), :],
                dst_ref=out_ref.at[pl.ds(send_c * chunk, chunk), :],
                send_sem=send_sems.at[n_hops + hop],
                recv_sem=recv_sems.at[n_hops + hop],
                device_id=(right,),
                device_id_type=pl.DeviceIdType.MESH,
            )
            rdma.start()
            rdma.wait()

    return pl.pallas_call(
        body,
        out_shape=jax.ShapeDtypeStruct((n, h), jnp.float32),
        in_specs=[
            pl.BlockSpec(memory_space=pltpu.VMEM),
            pl.BlockSpec(memory_space=pltpu.VMEM),
            pl.BlockSpec(memory_space=pltpu.VMEM),
            pl.BlockSpec(memory_space=pltpu.VMEM),
        ],
        out_specs=pl.BlockSpec(memory_space=pltpu.VMEM),
        scratch_shapes=[
            pltpu.VMEM((n_hops, chunk, h), jnp.float32),
            pltpu.SemaphoreType.DMA((2 * n_hops,)),
            pltpu.SemaphoreType.DMA((2 * n_hops,)),
        ],
        compiler_params=pltpu.CompilerParams(collective_id=0),
    )(x, router_W, route_idx, expert_W)


# device time: 60046 ns/iter; 1.6580x vs baseline; 1.6580x over previous
import jax
import jax.numpy as jnp
from jax import lax
from jax.experimental import pallas as pl
from jax.experimental.pallas import tpu as pltpu

N_DEV = 4


def kernel(x, router_W, route_idx, expert_W):
    n, d = x.shape
    n_loc, _, h = expert_W.shape
    n_exp = N_DEV * n_loc
    chunk = n // N_DEV
    hl = h // 2
    n_hops = N_DEV - 1

    def body(x_ref, rw_ref, idx_ref, ew_ref, out_ref,
             comm_cw, comm_ccw, send_cw, recv_cw, send_ccw, recv_ccw):
        my = lax.axis_index("i")
        left = (my - 1) % N_DEV
        right = (my + 1) % N_DEV

        barrier_sem = pltpu.get_barrier_semaphore()
        for nbr in (left, right):
            pl.semaphore_signal(
                barrier_sem, inc=1,
                device_id=(nbr,), device_id_type=pl.DeviceIdType.MESH,
            )
        pl.semaphore_wait(barrier_sem, 2)

        gids = my * n_loc + lax.broadcasted_iota(jnp.int32, (1, n_loc), 1)

        def compute_chunk(c):
            rows = pl.ds(c * chunk, chunk)
            xs = x_ref[rows, :]
            scores = jnp.dot(xs, rw_ref[:, :],
                             preferred_element_type=jnp.float32)
            m = jnp.max(scores, axis=-1, keepdims=True)
            p = jnp.exp(scores - m)
            p = p / jnp.sum(p, axis=-1, keepdims=True)
            idx0 = idx_ref[rows, 0:1]
            idx1 = idx_ref[rows, 1:2]
            e_iota = lax.broadcasted_iota(jnp.int32, (chunk, n_exp), 1)
            g0 = jnp.sum(jnp.where(idx0 == e_iota, p, 0.0),
                         axis=-1, keepdims=True)
            g1 = jnp.sum(jnp.where(idx1 == e_iota, p, 0.0),
                         axis=-1, keepdims=True)
            gs = g0 + g1
            ws = (jnp.where(idx0 == gids, g0 / gs, 0.0)
                  + jnp.where(idx1 == gids, g1 / gs, 0.0))
            acc = jnp.zeros((chunk, h), jnp.float32)
            for le in range(n_loc):
                acc = acc + jnp.dot(xs * ws[:, le:le + 1], ew_ref[le, :, :],
                                    preferred_element_type=jnp.float32)
            out_ref[rows, :] = acc

        def rs_rdma(hop, c, cw):
            rows = pl.ds(c * chunk, chunk)
            if cw:
                return pltpu.make_async_remote_copy(
                    src_ref=out_ref.at[rows, pl.ds(0, hl)],
                    dst_ref=comm_cw.at[hop],
                    send_sem=send_cw.at[hop], recv_sem=recv_cw.at[hop],
                    device_id=(right,), device_id_type=pl.DeviceIdType.MESH,
                )
            return pltpu.make_async_remote_copy(
                src_ref=out_ref.at[rows, pl.ds(hl, hl)],
                dst_ref=comm_ccw.at[hop],
                send_sem=send_ccw.at[hop], recv_sem=recv_ccw.at[hop],
                device_id=(left,), device_id_type=pl.DeviceIdType.MESH,
            )

        def rs_accum(hop, c, cw):
            rows = pl.ds(c * chunk, chunk)
            if cw:
                out_ref[rows, pl.ds(0, hl)] = (
                    out_ref[rows, pl.ds(0, hl)] + comm_cw[hop])
            else:
                out_ref[rows, pl.ds(hl, hl)] = (
                    out_ref[rows, pl.ds(hl, hl)] + comm_ccw[hop])

        compute_chunk(my)
        cw0 = rs_rdma(0, my, True)
        ccw0 = rs_rdma(0, my, False)
        cw0.start()
        ccw0.start()

        compute_chunk((my - 1) % N_DEV)
        cw0.wait()
        rs_accum(0, (my - 1) % N_DEV, True)
        cw1 = rs_rdma(1, (my - 1) % N_DEV, True)
        cw1.start()

        compute_chunk((my + 1) % N_DEV)
        ccw0.wait()
        rs_accum(0, (my + 1) % N_DEV, False)
        ccw1 = rs_rdma(1, (my + 1) % N_DEV, False)
        ccw1.start()

        compute_chunk((my + 2) % N_DEV)
        cw1.wait()
        rs_accum(1, (my - 2) % N_DEV, True)
        cw2 = rs_rdma(2, (my - 2) % N_DEV, True)
        cw2.start()
        ccw1.wait()
        rs_accum(1, (my + 2) % N_DEV, False)
        ccw2 = rs_rdma(2, (my + 2) % N_DEV, False)
        ccw2.start()

        cw2.wait()
        rs_accum(2, (my + 1) % N_DEV, True)
        ccw2.wait()
        rs_accum(2, (my - 1) % N_DEV, False)

        for hop in range(n_hops):
            c_cw = (my + 1 - hop) % N_DEV
            c_ccw = (my - 1 + hop) % N_DEV
            rows_cw = pl.ds(c_cw * chunk, chunk)
            rows_ccw = pl.ds(c_ccw * chunk, chunk)
            ag_cw = pltpu.make_async_remote_copy(
                src_ref=out_ref.at[rows_cw, pl.ds(0, hl)],
                dst_ref=out_ref.at[rows_cw, pl.ds(0, hl)],
                send_sem=send_cw.at[n_hops + hop],
                recv_sem=recv_cw.at[n_hops + hop],
                device_id=(right,), device_id_type=pl.DeviceIdType.MESH,
            )
            ag_ccw = pltpu.make_async_remote_copy(
                src_ref=out_ref.at[rows_ccw, pl.ds(hl, hl)],
                dst_ref=out_ref.at[rows_ccw, pl.ds(hl, hl)],
                send_sem=send_ccw.at[n_hops + hop],
                recv_sem=recv_ccw.at[n_hops + hop],
                device_id=(left,), device_id_type=pl.DeviceIdType.MESH,
            )
            ag_cw.start()
            ag_ccw.start()
            ag_cw.wait()
            ag_ccw.wait()

    return pl.pallas_call(
        body,
        out_shape=jax.ShapeDtypeStruct((n, h), jnp.float32),
        in_specs=[
            pl.BlockSpec(memory_space=pltpu.VMEM),
            pl.BlockSpec(memory_space=pltpu.VMEM),
            pl.BlockSpec(memory_space=pltpu.VMEM),
            pl.BlockSpec(memory_space=pltpu.VMEM),
        ],
        out_specs=pl.BlockSpec(memory_space=pltpu.VMEM),
        scratch_shapes=[
            pltpu.VMEM((N_DEV - 1, chunk, hl), jnp.float32),
            pltpu.VMEM((N_DEV - 1, chunk, hl), jnp.float32),
            pltpu.SemaphoreType.DMA((2 * (N_DEV - 1),)),
            pltpu.SemaphoreType.DMA((2 * (N_DEV - 1),)),
            pltpu.SemaphoreType.DMA((2 * (N_DEV - 1),)),
            pltpu.SemaphoreType.DMA((2 * (N_DEV - 1),)),
        ],
        compiler_params=pltpu.CompilerParams(collective_id=0),
    )(x, router_W, route_idx, expert_W)


# device time: 43400 ns/iter; 2.2939x vs baseline; 1.3835x over previous
import jax
import jax.numpy as jnp
from jax import lax
from jax.experimental import pallas as pl
from jax.experimental.pallas import tpu as pltpu

N_DEV = 4


def kernel(x, router_W, route_idx, expert_W):
    n, d = x.shape
    n_loc, _, h = expert_W.shape
    n_exp = N_DEV * n_loc
    chunk = n // N_DEV
    hl = h // 2
    n_hops = N_DEV - 1

    def body(x_ref, rw_ref, idx_ref, ew_ref, out_ref,
             comm_cw, comm_ccw, ag_cw, ag_ccw, stage_cw, stage_ccw,
             send_cw, recv_cw, send_ccw, recv_ccw):
        my = lax.axis_index("i")
        left = (my - 1) % N_DEV
        right = (my + 1) % N_DEV

        def rows(c):
            return pl.ds((c % N_DEV) * chunk, chunk)

        cw_cols = pl.ds(0, hl)
        ccw_cols = pl.ds(hl, hl)

        barrier_sem = pltpu.get_barrier_semaphore()
        for nbr in (left, right):
            pl.semaphore_signal(
                barrier_sem, inc=1,
                device_id=(nbr,), device_id_type=pl.DeviceIdType.MESH,
            )
        pl.semaphore_wait(barrier_sem, 2)

        gids = my * n_loc + lax.broadcasted_iota(jnp.int32, (1, n_loc), 1)

        def compute_chunk(c):
            r = rows(c)
            xs = x_ref[r, :]
            scores = jnp.dot(xs, rw_ref[:, :],
                             preferred_element_type=jnp.float32)
            m = jnp.max(scores, axis=-1, keepdims=True)
            p = jnp.exp(scores - m)
            p = p / jnp.sum(p, axis=-1, keepdims=True)
            idx0 = idx_ref[r, 0:1]
            idx1 = idx_ref[r, 1:2]
            e_iota = lax.broadcasted_iota(jnp.int32, (chunk, n_exp), 1)
            g0 = jnp.sum(jnp.where(idx0 == e_iota, p, 0.0),
                         axis=-1, keepdims=True)
            g1 = jnp.sum(jnp.where(idx1 == e_iota, p, 0.0),
                         axis=-1, keepdims=True)
            gs = g0 + g1
            ws = (jnp.where(idx0 == gids, g0 / gs, 0.0)
                  + jnp.where(idx1 == gids, g1 / gs, 0.0))
            acc = jnp.zeros((chunk, h), jnp.float32)
            for le in range(n_loc):
                acc = acc + jnp.dot(xs * ws[:, le:le + 1], ew_ref[le, :, :],
                                    preferred_element_type=jnp.float32)
            out_ref[r, :] = acc

        def rs_send(hop, cw):
            if cw:
                rdma = pltpu.make_async_remote_copy(
                    src_ref=stage_cw, dst_ref=comm_cw.at[hop],
                    send_sem=send_cw.at[hop], recv_sem=recv_cw.at[hop],
                    device_id=(right,), device_id_type=pl.DeviceIdType.MESH,
                )
            else:
                rdma = pltpu.make_async_remote_copy(
                    src_ref=stage_ccw, dst_ref=comm_ccw.at[hop],
                    send_sem=send_ccw.at[hop], recv_sem=recv_ccw.at[hop],
                    device_id=(left,), device_id_type=pl.DeviceIdType.MESH,
                )
            rdma.start()
            return rdma

        def rs_accum(hop, c, cw):
            r = rows(c)
            if cw:
                t = out_ref[r, cw_cols] + comm_cw[hop].astype(jnp.float32)
                out_ref[r, cw_cols] = t
                stage_cw[:, :] = t.astype(jnp.bfloat16)
            else:
                t = out_ref[r, ccw_cols] + comm_ccw[hop].astype(jnp.float32)
                out_ref[r, ccw_cols] = t
                stage_ccw[:, :] = t.astype(jnp.bfloat16)

        compute_chunk(my)
        stage_cw[:, :] = out_ref[rows(my), cw_cols].astype(jnp.bfloat16)
        stage_ccw[:, :] = out_ref[rows(my), ccw_cols].astype(jnp.bfloat16)
        cw0 = rs_send(0, True)
        ccw0 = rs_send(0, False)

        compute_chunk(my - 1)
        cw0.wait()
        rs_accum(0, my - 1, True)
        cw1 = rs_send(1, True)

        compute_chunk(my + 1)
        ccw0.wait()
        rs_accum(0, my + 1, False)
        ccw1 = rs_send(1, False)

        compute_chunk(my + 2)
        cw1.wait()
        rs_accum(1, my - 2, True)
        cw2 = rs_send(2, True)
        ccw1.wait()
        rs_accum(1, my + 2, False)
        ccw2 = rs_send(2, False)

        cw2.wait()
        rs_accum(2, my + 1, True)
        ccw2.wait()
        rs_accum(2, my - 1, False)

        def ag_send(hop, cw):
            if cw:
                rdma = pltpu.make_async_remote_copy(
                    src_ref=stage_cw if hop == 0 else ag_cw.at[hop - 1],
                    dst_ref=ag_cw.at[hop],
                    send_sem=send_cw.at[n_hops + hop],
                    recv_sem=recv_cw.at[n_hops + hop],
                    device_id=(right,), device_id_type=pl.DeviceIdType.MESH,
                )
            else:
                rdma = pltpu.make_async_remote_copy(
                    src_ref=stage_ccw if hop == 0 else ag_ccw.at[hop - 1],
                    dst_ref=ag_ccw.at[hop],
                    send_sem=send_ccw.at[n_hops + hop],
                    recv_sem=recv_ccw.at[n_hops + hop],
                    device_id=(left,), device_id_type=pl.DeviceIdType.MESH,
                )
            rdma.start()
            return rdma

        agc = ag_send(0, True)
        agg = ag_send(0, False)
        for hop in range(n_hops):
            agc.wait()
            if hop + 1 < n_hops:
                agc_next = ag_send(hop + 1, True)
            out_ref[rows(my - hop), cw_cols] = ag_cw[hop].astype(jnp.float32)
            agg.wait()
            if hop + 1 < n_hops:
                agg_next = ag_send(hop + 1, False)
            out_ref[rows(my + hop), ccw_cols] = ag_ccw[hop].astype(jnp.float32)
            if hop + 1 < n_hops:
                agc, agg = agc_next, agg_next

    bf = jnp.bfloat16
    return pl.pallas_call(
        body,
        out_shape=jax.ShapeDtypeStruct((n, h), jnp.float32),
        in_specs=[
            pl.BlockSpec(memory_space=pltpu.VMEM),
            pl.BlockSpec(memory_space=pltpu.VMEM),
            pl.BlockSpec(memory_space=pltpu.VMEM),
            pl.BlockSpec(memory_space=pltpu.VMEM),
        ],
        out_specs=pl.BlockSpec(memory_space=pltpu.VMEM),
        scratch_shapes=[
            pltpu.VMEM((n_hops, chunk, hl), bf),
            pltpu.VMEM((n_hops, chunk, hl), bf),
            pltpu.VMEM((n_hops, chunk, hl), bf),
            pltpu.VMEM((n_hops, chunk, hl), bf),
            pltpu.VMEM((chunk, hl), bf),
            pltpu.VMEM((chunk, hl), bf),
            pltpu.SemaphoreType.DMA((2 * n_hops,)),
            pltpu.SemaphoreType.DMA((2 * n_hops,)),
            pltpu.SemaphoreType.DMA((2 * n_hops,)),
            pltpu.SemaphoreType.DMA((2 * n_hops,)),
        ],
        compiler_params=pltpu.CompilerParams(collective_id=0),
    )(x, router_W, route_idx, expert_W)


# device time: 43035 ns/iter; 2.3134x vs baseline; 1.0085x over previous
import jax
import jax.numpy as jnp
from jax import lax
from jax.experimental import pallas as pl
from jax.experimental.pallas import tpu as pltpu

N_DEV = 4


def kernel(x, router_W, route_idx, expert_W):
    n, d = x.shape
    n_loc, _, h = expert_W.shape
    n_exp = N_DEV * n_loc
    chunk = n // N_DEV
    hl = h // 2
    n_hops = N_DEV - 1

    def body(x_ref, rw_ref, idx_ref, ew_ref, out_ref,
             comm_cw, comm_ccw, ag_cw, ag_ccw, part_cw, part_ccw,
             send_cw, recv_cw, send_ccw, recv_ccw):
        my = lax.axis_index("i")
        left = (my - 1) % N_DEV
        right = (my + 1) % N_DEV

        def rows(c):
            return pl.ds((c % N_DEV) * chunk, chunk)

        cw_cols = pl.ds(0, hl)
        ccw_cols = pl.ds(hl, hl)

        barrier_sem = pltpu.get_barrier_semaphore()
        for nbr in (left, right):
            pl.semaphore_signal(
                barrier_sem, inc=1,
                device_id=(nbr,), device_id_type=pl.DeviceIdType.MESH,
            )
        pl.semaphore_wait(barrier_sem, 2)

        gids = my * n_loc + lax.broadcasted_iota(jnp.int32, (1, n_loc), 1)

        def compute_chunk(c):
            r = rows(c)
            xs = x_ref[r, :]
            scores = jnp.dot(xs, rw_ref[:, :],
                             preferred_element_type=jnp.float32)
            m = jnp.max(scores, axis=-1, keepdims=True)
            p = jnp.exp(scores - m)
            p = p / jnp.sum(p, axis=-1, keepdims=True)
            idx0 = idx_ref[r, 0:1]
            idx1 = idx_ref[r, 1:2]
            e_iota = lax.broadcasted_iota(jnp.int32, (chunk, n_exp), 1)
            g0 = jnp.sum(jnp.where(idx0 == e_iota, p, 0.0),
                         axis=-1, keepdims=True)
            g1 = jnp.sum(jnp.where(idx1 == e_iota, p, 0.0),
                         axis=-1, keepdims=True)
            gs = g0 + g1
            ws = (jnp.where(idx0 == gids, g0 / gs, 0.0)
                  + jnp.where(idx1 == gids, g1 / gs, 0.0))
            acc = jnp.zeros((chunk, h), jnp.float32)
            for le in range(n_loc):
                acc = acc + jnp.dot(xs * ws[:, le:le + 1], ew_ref[le, :, :],
                                    preferred_element_type=jnp.float32)
            part_cw[r, :] = acc[:, :hl].astype(jnp.bfloat16)
            part_ccw[r, :] = acc[:, hl:].astype(jnp.bfloat16)

        def rs_send(hop, cw):
            if cw:
                rdma = pltpu.make_async_remote_copy(
                    src_ref=(part_cw.at[rows(my), :] if hop == 0
                             else comm_cw.at[hop - 1]),
                    dst_ref=comm_cw.at[hop],
                    send_sem=send_cw.at[hop], recv_sem=recv_cw.at[hop],
                    device_id=(right,), device_id_type=pl.DeviceIdType.MESH,
                )
            else:
                rdma = pltpu.make_async_remote_copy(
                    src_ref=(part_ccw.at[rows(my), :] if hop == 0
                             else comm_ccw.at[hop - 1]),
                    dst_ref=comm_ccw.at[hop],
                    send_sem=send_ccw.at[hop], recv_sem=recv_ccw.at[hop],
                    device_id=(left,), device_id_type=pl.DeviceIdType.MESH,
                )
            rdma.start()
            return rdma

        def rs_accum(hop, c, cw):
            if cw:
                comm_cw[hop, :, :] = comm_cw[hop] + part_cw[rows(c), :]
            else:
                comm_ccw[hop, :, :] = comm_ccw[hop] + part_ccw[rows(c), :]

        compute_chunk(my)
        cw0 = rs_send(0, True)
        ccw0 = rs_send(0, False)

        compute_chunk(my - 1)
        cw0.wait()
        rs_accum(0, my - 1, True)
        cw1 = rs_send(1, True)

        compute_chunk(my + 1)
        ccw0.wait()
        rs_accum(0, my + 1, False)
        ccw1 = rs_send(1, False)

        compute_chunk(my + 2)
        cw1.wait()
        rs_accum(1, my - 2, True)
        cw2 = rs_send(2, True)
        ccw1.wait()
        rs_accum(1, my + 2, False)
        ccw2 = rs_send(2, False)

        def ag_send(hop, cw):
            if cw:
                rdma = pltpu.make_async_remote_copy(
                    src_ref=comm_cw.at[2] if hop == 0 else ag_cw.at[hop - 1],
                    dst_ref=ag_cw.at[hop],
                    send_sem=send_cw.at[n_hops + hop],
                    recv_sem=recv_cw.at[n_hops + hop],
                    device_id=(right,), device_id_type=pl.DeviceIdType.MESH,
                )
            else:
                rdma = pltpu.make_async_remote_copy(
                    src_ref=comm_ccw.at[2] if hop == 0 else ag_ccw.at[hop - 1],
                    dst_ref=ag_ccw.at[hop],
                    send_sem=send_ccw.at[n_hops + hop],
                    recv_sem=recv_ccw.at[n_hops + hop],
                    device_id=(left,), device_id_type=pl.DeviceIdType.MESH,
                )
            rdma.start()
            return rdma

        cw2.wait()
        rs_accum(2, my + 1, True)
        agc = ag_send(0, True)
        out_ref[rows(my + 1), cw_cols] = comm_cw[2].astype(jnp.float32)

        ccw2.wait()
        rs_accum(2, my - 1, False)
        agg = ag_send(0, False)
        out_ref[rows(my - 1), ccw_cols] = comm_ccw[2].astype(jnp.float32)

        for hop in range(n_hops):
            agc.wait()
            if hop + 1 < n_hops:
                agc_next = ag_send(hop + 1, True)
            out_ref[rows(my - hop), cw_cols] = ag_cw[hop].astype(jnp.float32)
            agg.wait()
            if hop + 1 < n_hops:
                agg_next = ag_send(hop + 1, False)
            out_ref[rows(my + hop), ccw_cols] = ag_ccw[hop].astype(jnp.float32)
            if hop + 1 < n_hops:
                agc, agg = agc_next, agg_next

    bf = jnp.bfloat16
    return pl.pallas_call(
        body,
        out_shape=jax.ShapeDtypeStruct((n, h), jnp.float32),
        in_specs=[
            pl.BlockSpec(memory_space=pltpu.VMEM),
            pl.BlockSpec(memory_space=pltpu.VMEM),
            pl.BlockSpec(memory_space=pltpu.VMEM),
            pl.BlockSpec(memory_space=pltpu.VMEM),
        ],
        out_specs=pl.BlockSpec(memory_space=pltpu.VMEM),
        scratch_shapes=[
            pltpu.VMEM((n_hops, chunk, hl), bf),
            pltpu.VMEM((n_hops, chunk, hl), bf),
            pltpu.VMEM((n_hops, chunk, hl), bf),
            pltpu.VMEM((n_hops, chunk, hl), bf),
            pltpu.VMEM((n, hl), bf),
            pltpu.VMEM((n, hl), bf),
            pltpu.SemaphoreType.DMA((2 * n_hops,)),
            pltpu.SemaphoreType.DMA((2 * n_hops,)),
            pltpu.SemaphoreType.DMA((2 * n_hops,)),
            pltpu.SemaphoreType.DMA((2 * n_hops,)),
        ],
        compiler_params=pltpu.CompilerParams(collective_id=0),
    )(x, router_W, route_idx, expert_W)


# device time: 35361 ns/iter; 2.8154x vs baseline; 1.2170x over previous
import jax
import jax.numpy as jnp
from jax import lax
from jax.experimental import pallas as pl
from jax.experimental.pallas import tpu as pltpu

N_DEV = 4
NS = 4
N_TR = 12


def kernel(x, router_W, route_idx, expert_W):
    n, d = x.shape
    n_loc, _, h = expert_W.shape
    n_exp = N_DEV * n_loc
    chunk = n // N_DEV
    hl = h // 2
    hsq = hl // NS

    def body(x_ref, rw_ref, idx_ref, ew_ref, out_ref,
             part_L, part_R, dL, dR, fL, fR, cL, cR,
             send_sems, recv_sems):
        my = lax.axis_index("i")
        left = (my - 1) % N_DEV
        right = (my + 1) % N_DEV

        def rows(c):
            return pl.ds((c % N_DEV) * chunk, chunk)

        def scols(s):
            return pl.ds(s * hsq, hsq)

        def ocols(s, is_L):
            return pl.ds((0 if is_L else hl) + s * hsq, hsq)

        barrier_sem = pltpu.get_barrier_semaphore()
        for nbr in (left, right):
            pl.semaphore_signal(
                barrier_sem, inc=1,
                device_id=(nbr,), device_id_type=pl.DeviceIdType.MESH,
            )
        pl.semaphore_wait(barrier_sem, 2)

        gids = my * n_loc + lax.broadcasted_iota(jnp.int32, (1, n_loc), 1)

        def compute_chunk(c):
            r = rows(c)
            xs = x_ref[r, :]
            scores = jnp.dot(xs, rw_ref[:, :],
                             preferred_element_type=jnp.float32)
            m = jnp.max(scores, axis=-1, keepdims=True)
            p = jnp.exp(scores - m)
            p = p / jnp.sum(p, axis=-1, keepdims=True)
            idx0 = idx_ref[r, 0:1]
            idx1 = idx_ref[r, 1:2]
            e_iota = lax.broadcasted_iota(jnp.int32, (chunk, n_exp), 1)
            g0 = jnp.sum(jnp.where(idx0 == e_iota, p, 0.0),
                         axis=-1, keepdims=True)
            g1 = jnp.sum(jnp.where(idx1 == e_iota, p, 0.0),
                         axis=-1, keepdims=True)
            gs = g0 + g1
            ws = (jnp.where(idx0 == gids, g0 / gs, 0.0)
                  + jnp.where(idx1 == gids, g1 / gs, 0.0))
            acc = jnp.zeros((chunk, h), jnp.float32)
            for le in range(n_loc):
                acc = acc + jnp.dot(xs * ws[:, le:le + 1], ew_ref[le, :, :],
                                    preferred_element_type=jnp.float32)
            part_L[r, :] = acc[:, :hl].astype(jnp.bfloat16)
            part_R[r, :] = acc[:, hl:].astype(jnp.bfloat16)

        def xfer(k, s, src, dst, dev):
            rdma = pltpu.make_async_remote_copy(
                src_ref=src, dst_ref=dst,
                send_sem=send_sems.at[s, k], recv_sem=recv_sems.at[s, k],
                device_id=(dev,), device_id_type=pl.DeviceIdType.MESH,
            )
            rdma.start()
            return rdma

        compute_chunk(my + 1)
        r_dirR = [xfer(0, s, part_R.at[rows(my + 1), scols(s)],
                       dR.at[:, scols(s)], right) for s in range(NS)]

        compute_chunk(my - 1)
        r_dirL = [xfer(1, s, part_L.at[rows(my - 1), scols(s)],
                       dL.at[:, scols(s)], left) for s in range(NS)]

        compute_chunk(my + 2)
        r_feedL = [xfer(2, s, part_L.at[rows(my + 2), scols(s)],
                        fL.at[:, scols(s)], right) for s in range(NS)]
        r_feedR = [xfer(3, s, part_R.at[rows(my + 2), scols(s)],
                        fR.at[:, scols(s)], left) for s in range(NS)]

        compute_chunk(my)

        r_combL = []
        for s in range(NS):
            r_feedL[s].wait()
            fL[:, scols(s)] = fL[:, scols(s)] + part_L[rows(my + 1), scols(s)]
            r_combL.append(xfer(4, s, fL.at[:, scols(s)],
                                cL.at[:, scols(s)], right))
        r_combR = []
        for s in range(NS):
            r_feedR[s].wait()
            fR[:, scols(s)] = fR[:, scols(s)] + part_R[rows(my - 1), scols(s)]
            r_combR.append(xfer(5, s, fR.at[:, scols(s)],
                                cR.at[:, scols(s)], left))

        ag = {}
        for s in range(NS):
            r_dirL[s].wait()
            r_combL[s].wait()
            tL = part_L[rows(my), scols(s)] + dL[:, scols(s)] + cL[:, scols(s)]
            out_ref[rows(my), ocols(s, True)] = tL
            ag[(6, s)] = xfer(6, s, out_ref.at[rows(my), ocols(s, True)],
                              out_ref.at[rows(my), ocols(s, True)], right)
            ag[(8, s)] = xfer(8, s, out_ref.at[rows(my), ocols(s, True)],
                              out_ref.at[rows(my), ocols(s, True)], left)
        for s in range(NS):
            r_dirR[s].wait()
            r_combR[s].wait()
            tR = part_R[rows(my), scols(s)] + dR[:, scols(s)] + cR[:, scols(s)]
            out_ref[rows(my), ocols(s, False)] = tR
            ag[(7, s)] = xfer(7, s, out_ref.at[rows(my), ocols(s, False)],
                              out_ref.at[rows(my), ocols(s, False)], right)
            ag[(9, s)] = xfer(9, s, out_ref.at[rows(my), ocols(s, False)],
                              out_ref.at[rows(my), ocols(s, False)], left)

        fwd = {}
        for s in range(NS):
            ag[(6, s)].wait()
            fwd[(10, s)] = xfer(10, s,
                                out_ref.at[rows(my - 1), ocols(s, True)],
                                out_ref.at[rows(my - 1), ocols(s, True)],
                                right)
        for s in range(NS):
            ag[(9, s)].wait()
            fwd[(11, s)] = xfer(11, s,
                                out_ref.at[rows(my + 1), ocols(s, False)],
                                out_ref.at[rows(my + 1), ocols(s, False)],
                                left)
        for s in range(NS):
            ag[(7, s)].wait()
            ag[(8, s)].wait()
        for s in range(NS):
            fwd[(10, s)].wait()
            fwd[(11, s)].wait()

    bf = jnp.bfloat16
    return pl.pallas_call(
        body,
        out_shape=jax.ShapeDtypeStruct((n, h), bf),
        in_specs=[
            pl.BlockSpec(memory_space=pltpu.VMEM),
            pl.BlockSpec(memory_space=pltpu.VMEM),
            pl.BlockSpec(memory_space=pltpu.VMEM),
            pl.BlockSpec(memory_space=pltpu.VMEM),
        ],
        out_specs=pl.BlockSpec(memory_space=pltpu.VMEM),
        scratch_shapes=[
            pltpu.VMEM((n, hl), bf),
            pltpu.VMEM((n, hl), bf),
            pltpu.VMEM((chunk, hl), bf),
            pltpu.VMEM((chunk, hl), bf),
            pltpu.VMEM((chunk, hl), bf),
            pltpu.VMEM((chunk, hl), bf),
            pltpu.VMEM((chunk, hl), bf),
            pltpu.VMEM((chunk, hl), bf),
            pltpu.SemaphoreType.DMA((NS, N_TR)),
            pltpu.SemaphoreType.DMA((NS, N_TR)),
        ],
        compiler_params=pltpu.CompilerParams(collective_id=0),
    )(x, router_W, route_idx, expert_W)
